# Initial kernel scaffold; baseline (speedup 1.0000x reference)
import jax
import jax.numpy as jnp
from jax.experimental import pallas as pl


def kernel(x, edge_index, edge_weight, ln1_g, ln1_b, W1, ln2_g, ln2_b, W2, ln3_g, ln3_b, W3):
    raise NotImplementedError("write your pallas kernel here")


# speedup vs baseline: 1.2254x; 1.2254x over previous
import functools

import jax
import jax.numpy as jnp
from jax import lax
from jax.experimental import pallas as pl
from jax.experimental.pallas import tpu as pltpu
from jax.experimental.pallas import tpu_sc as plsc

NC = 2
NS = 16
NW = NC * NS
L = 16

N = 10000
E = 320000
H = 128

PN = 320
NPAD = NW * PN

EPW = E // NW
CB = 80
NCH_A = EPW // CB

CH = 4000
NCH_C = E // CH
NST = CH // L
Q = 32


def _gather_body(x_hbm, src_hbm, dst_hbm, w_hbm, a1_hbm, a2_hbm,
                 sbuf, dbuf, wbuf, xi_v, xj_v, sem1, sem2):
  wid = lax.axis_index("s") * NC + lax.axis_index("c")
  base = wid * EPW

  def chunk(ci, carry):
    off = base + ci * CB
    pltpu.sync_copy(src_hbm.at[pl.ds(off, CB)], sbuf)
    pltpu.sync_copy(dst_hbm.at[pl.ds(off, CB)], dbuf)
    pltpu.sync_copy(w_hbm.at[pl.ds(off, CB)], wbuf.at[pl.ds(0, CB)])
    pltpu.async_copy(x_hbm.at[dbuf], xi_v, sem1).wait()
    pltpu.async_copy(x_hbm.at[sbuf], xj_v, sem2).wait()
    pltpu.sync_copy(xi_v, a1_hbm.at[pl.ds(off, CB)])

    def edge(e, c2):
      we = wbuf[pl.ds(e, L)][0]
      for t in range(H // L):
        sl = pl.ds(t * L, L)
        xj_v[e, sl] = (xj_v[e, sl] - xi_v[e, sl]) * we
      return c2

    lax.fori_loop(0, CB, edge, 0)
    pltpu.sync_copy(xj_v, a2_hbm.at[pl.ds(off, CB)])
    return carry

  lax.fori_loop(0, NCH_A, chunk, 0)


def _sc_gather(x, src, dst, w):
  mesh = plsc.VectorSubcoreMesh(core_axis_name="c", subcore_axis_name="s")
  f = pl.kernel(
      _gather_body,
      out_type=(
          jax.ShapeDtypeStruct((E, H), jnp.float32),
          jax.ShapeDtypeStruct((E, H), jnp.float32),
      ),
      mesh=mesh,
      scratch_types=[
          pltpu.VMEM((CB,), jnp.int32),
          pltpu.VMEM((CB,), jnp.int32),
          pltpu.VMEM((CB + L,), jnp.float32),
          pltpu.VMEM((CB, H), jnp.float32),
          pltpu.VMEM((CB, H), jnp.float32),
          pltpu.SemaphoreType.DMA,
          pltpu.SemaphoreType.DMA,
      ],
  )
  return f(x, src, dst, w)


def _ln_lrelu(h, g, b):
  mu = jnp.mean(h, axis=1, keepdims=True)
  var = jnp.mean((h - mu) ** 2, axis=1, keepdims=True)
  hn = (h - mu) * lax.rsqrt(var + 1e-5) * g + b
  return jnp.where(hn >= 0, hn, 0.2 * hn)


def _mlp_body(a1_ref, a2_ref, g1, b1, w1, g2, b2, w2, g3, b3, w3, o_ref):
  h = jnp.concatenate([a1_ref[...], a2_ref[...]], axis=1)
  h = jnp.dot(_ln_lrelu(h, g1[...], b1[...]), w1[...],
              preferred_element_type=jnp.float32)
  h = jnp.dot(_ln_lrelu(h, g2[...], b2[...]), w2[...],
              preferred_element_type=jnp.float32)
  h = jnp.dot(_ln_lrelu(h, g3[...], b3[...]), w3[...],
              preferred_element_type=jnp.float32)
  o_ref[...] = h


def _tc_mlp(a1, a2, g1, b1, w1, g2, b2, w2, g3, b3, w3):
  be = 512
  nb = E // be
  full = lambda shape: pl.BlockSpec(shape, lambda i: (0, 0))
  return pl.pallas_call(
      _mlp_body,
      grid=(nb,),
      in_specs=[
          pl.BlockSpec((be, H), lambda i: (i, 0)),
          pl.BlockSpec((be, H), lambda i: (i, 0)),
          full((1, 2 * H)), full((1, 2 * H)), full((2 * H, H)),
          full((1, H)), full((1, H)), full((H, H)),
          full((1, H)), full((1, H)), full((H, H)),
      ],
      out_specs=pl.BlockSpec((be, H), lambda i: (i, 0)),
      out_shape=jax.ShapeDtypeStruct((E, H), jnp.float32),
      compiler_params=pltpu.CompilerParams(
          dimension_semantics=("arbitrary",)),
  )(a1, a2, g1.reshape(1, -1), b1.reshape(1, -1), w1,
    g2.reshape(1, -1), b2.reshape(1, -1), w2,
    g3.reshape(1, -1), b3.reshape(1, -1), w3)


def _segmax_body(h_hbm, dst_hbm, x_hbm, o_hbm,
                 dbuf, cnts, locid, ldst, rows, acc, xrows, sem):
  wid = lax.axis_index("s") * NC + lax.axis_index("c")
  lo = wid * PN
  neg_inf = jnp.full((L,), -jnp.inf, jnp.float32)

  def initr(r, c):
    for t in range(H // L):
      acc[r, pl.ds(t * L, L)] = neg_inf
    return c

  lax.fori_loop(0, PN, initr, 0)

  iota = lax.iota(jnp.int32, L)

  def chunk(ci, carry):
    cbase = ci * CH
    pltpu.sync_copy(dst_hbm.at[pl.ds(cbase, CH)], dbuf)

    def p1(k, c):
      v = dbuf[pl.ds(k * L, L)]
      msk = (v >= lo) & (v < lo + PN)
      cnts[k] = jnp.sum(msk.astype(jnp.int32))
      return c

    lax.fori_loop(0, NST, p1, 0)

    def p2(k, m):
      v = dbuf[pl.ds(k * L, L)]
      msk = (v >= lo) & (v < lo + PN)
      gid = iota + (cbase + k * L)
      plsc.store_compressed(locid.at[pl.ds(m, L)], gid, mask=msk)
      plsc.store_compressed(ldst.at[pl.ds(m, L)], v - lo, mask=msk)
      return m + cnts[k]

    m_tot = lax.fori_loop(0, NST, p2, jnp.int32(0))

    zz = jnp.zeros((L,), jnp.int32)
    locid[pl.ds(m_tot, L)] = zz
    locid[pl.ds(m_tot + L, L)] = zz

    nq = lax.shift_right_logical(m_tot + (Q - 1), 5)

    def quant(q, c):
      pltpu.async_copy(h_hbm.at[locid.at[pl.ds(q * Q, Q)]], rows, sem).wait()
      lim = jnp.minimum(Q, m_tot - q * Q)

      def upd(i, c2):
        r = ldst[pl.ds(q * Q + i, L)][0]
        for t in range(H // L):
          sl = pl.ds(t * L, L)
          acc[r, sl] = jnp.maximum(acc[r, sl], rows[i, sl])
        return c2

      lax.fori_loop(0, lim, upd, 0)
      return c

    lax.fori_loop(0, nq, quant, 0)
    return carry

  lax.fori_loop(0, NCH_C, chunk, 0)

  pltpu.sync_copy(x_hbm.at[pl.ds(lo, PN)], xrows)

  def fin(r, c):
    for t in range(H // L):
      sl = pl.ds(t * L, L)
      a = acc[r, sl]
      acc[r, sl] = jnp.where(a == -jnp.inf, 0.0, a) + xrows[r, sl]
    return c

  lax.fori_loop(0, PN, fin, 0)
  pltpu.sync_copy(acc, o_hbm.at[pl.ds(lo, PN)])


def _sc_segmax(h, dst, x_pad):
  mesh = plsc.VectorSubcoreMesh(core_axis_name="c", subcore_axis_name="s")
  f = pl.kernel(
      _segmax_body,
      out_type=jax.ShapeDtypeStruct((NPAD, H), jnp.float32),
      mesh=mesh,
      compiler_params=pltpu.CompilerParams(needs_layout_passes=False),
      scratch_types=[
          pltpu.VMEM((CH,), jnp.int32),
          pltpu.SMEM((NST,), jnp.int32),
          pltpu.VMEM((CH + 2 * L,), jnp.int32),
          pltpu.VMEM((CH + 2 * L,), jnp.int32),
          pltpu.VMEM((Q, H), jnp.float32),
          pltpu.VMEM((PN, H), jnp.float32),
          pltpu.VMEM((PN, H), jnp.float32),
          pltpu.SemaphoreType.DMA,
      ],
  )
  return f(h, dst, x_pad)


def kernel(x, edge_index, edge_weight, ln1_g, ln1_b, W1,
           ln2_g, ln2_b, W2, ln3_g, ln3_b, W3):
  src = edge_index[0]
  dst = edge_index[1]
  a1, a2 = _sc_gather(x, src, dst, edge_weight)
  h = _tc_mlp(a1, a2, ln1_g, ln1_b, W1, ln2_g, ln2_b, W2, ln3_g, ln3_b, W3)
  x_pad = jnp.pad(x, ((0, NPAD - N), (0, 0)))
  out = _sc_segmax(h, dst, x_pad)
  return out[:N]
